# baseline (speedup 1.0000x reference)
import functools
import jax
import jax.numpy as jnp
from jax import lax
from jax.experimental import pallas as pl
from jax.experimental.pallas import tpu as pltpu
from jax.experimental.pallas import tpu_sc as plsc

B = 4
N = 2
H = W = 384
P = H * W
TOT = B * N * P
NUM_BINS = 16
KR = 2
EPS = 1e-8

NC, NS, L = 2, 16, 16
NW = NC * NS
SPB = P // (NW // B)
NCHK = 6
CPX = SPB // NCHK
CVEC = CPX // L
UNROLL = 4


def _bsp(d):
    ad = jnp.abs(d)
    c1 = (0.5 * ad - 1.0) * (ad * ad) + (2.0 / 3.0)
    t = jnp.maximum(2.0 - ad, 0.0)
    c2 = t * t * t * (1.0 / 6.0)
    return jnp.where(ad < 1.0, c1, c2)


def _pixel(a0, a1):
    mn = jnp.minimum(a0, a1)
    bw = (jnp.maximum(a0, a1) - mn) * (1.0 / NUM_BINS)
    pmin = mn - KR * bw
    inv = 1.0 / jnp.maximum(bw, EPS)
    p0 = (a0 - pmin) * inv
    p1 = (a1 - pmin) * inv
    g0 = p0.astype(jnp.int32).astype(jnp.float32)
    g1 = p1.astype(jnp.int32).astype(jnp.float32)
    f0 = p0 - g0
    f1 = p1 - g1
    zero = jnp.zeros_like(a0)
    w0 = (0.5 * f0 - 1.0) * (f0 * f0) + (2.0 / 3.0)
    w1 = (0.5 * f1 - 1.0) * (f1 * f1) + (2.0 / 3.0)
    n0 = jnp.where(g0 >= 1.0, w0 + _bsp(p1 - g0), zero)
    n1 = jnp.where(g1 >= 1.0, w1 + _bsp(p0 - g1), zero)
    v0 = jnp.maximum(2.0 - p0, 0.0)
    v1 = jnp.maximum(2.0 - p1, 0.0)
    q0 = jnp.maximum(1.0 - p0, 0.0)
    q1 = jnp.maximum(1.0 - p1, 0.0)
    cv = v0 * v0 * v0 + v1 * v1 * v1
    cq = q0 * q0 * q0 + q1 * q1 * q1
    hsum = jnp.maximum(2.0 - cv * (1.0 / 6.0) + cq * 0.5, EPS)
    rec = 1.0 / hsum
    return n0 * rec, n1 * rec


@functools.lru_cache(maxsize=1)
def _build():
    mesh = plsc.VectorSubcoreMesh(core_axis_name="c", subcore_axis_name="s")

    @functools.partial(
        pl.kernel,
        mesh=mesh,
        out_type=jax.ShapeDtypeStruct((TOT,), jnp.float32),
        scratch_types=[
            pltpu.VMEM((SPB,), jnp.float32),
            pltpu.VMEM((SPB,), jnp.float32),
            pltpu.VMEM((SPB,), jnp.float32),
            pltpu.VMEM((SPB,), jnp.float32),
            pltpu.SemaphoreType.DMA,
            pltpu.SemaphoreType.DMA,
            pltpu.SemaphoreType.DMA,
        ],
    )
    def _sc_kernel(img_hbm, out_hbm, v0, v1, o0, o1, sem_a, sem_b, sem_out):
        wid = lax.axis_index("c") * NS + lax.axis_index("s")
        b = wid // (NW // B)
        s = wid % (NW // B)
        off0 = b * (N * P) + s * SPB
        off1 = off0 + P
        HALF = SPB // 2
        ha0 = pltpu.async_copy(img_hbm.at[pl.ds(off0, HALF)],
                               v0.at[pl.ds(0, HALF)], sem_a)
        ha1 = pltpu.async_copy(img_hbm.at[pl.ds(off1, HALF)],
                               v1.at[pl.ds(0, HALF)], sem_a)
        hb0 = pltpu.async_copy(img_hbm.at[pl.ds(off0 + HALF, HALF)],
                               v0.at[pl.ds(HALF, HALF)], sem_b)
        hb1 = pltpu.async_copy(img_hbm.at[pl.ds(off1 + HALF, HALF)],
                               v1.at[pl.ds(HALF, HALF)], sem_b)

        def _body(i, carry):
            for u in range(UNROLL):
                sl = pl.ds((i * UNROLL + u) * L, L)
                r0, r1 = _pixel(v0[sl], v1[sl])
                o0[sl] = r0
                o1[sl] = r1
            return carry

        NIT = SPB // L // UNROLL
        ha0.wait()
        ha1.wait()
        lax.fori_loop(0, NIT // 2, _body, 0)
        ho0 = pltpu.async_copy(o0.at[pl.ds(0, HALF)],
                               out_hbm.at[pl.ds(off0, HALF)], sem_out)
        ho1 = pltpu.async_copy(o1.at[pl.ds(0, HALF)],
                               out_hbm.at[pl.ds(off1, HALF)], sem_out)
        hb0.wait()
        hb1.wait()
        lax.fori_loop(NIT // 2, NIT, _body, 0)
        ho2 = pltpu.async_copy(o0.at[pl.ds(HALF, HALF)],
                               out_hbm.at[pl.ds(off0 + HALF, HALF)], sem_out)
        ho3 = pltpu.async_copy(o1.at[pl.ds(HALF, HALF)],
                               out_hbm.at[pl.ds(off1 + HALF, HALF)], sem_out)
        ho0.wait()
        ho1.wait()
        ho2.wait()
        ho3.wait()

    return _sc_kernel


def kernel(images):
    flat = images.reshape(TOT)
    out = _build()(flat)
    return out.reshape(B, N, H, W)


# speedup vs baseline: 1.0079x; 1.0079x over previous
import functools
import jax
import jax.numpy as jnp
from jax import lax
from jax.experimental import pallas as pl
from jax.experimental.pallas import tpu as pltpu
from jax.experimental.pallas import tpu_sc as plsc

B = 4
N = 2
H = W = 384
P = H * W
TOT = B * N * P
NUM_BINS = 16
KR = 2
EPS = 1e-8

NC, NS, L = 2, 16, 16
NW = NC * NS
SPB = P // (NW // B)
NCHK = 6
CPX = SPB // NCHK
CVEC = CPX // L
UNROLL = 2


def _bsp(d):
    ad = jnp.abs(d)
    s = jnp.maximum(2.0 - ad, 0.0)
    r = jnp.maximum(1.0 - ad, 0.0)
    return s * s * s * (1.0 / 6.0) - r * r * r * (2.0 / 3.0)


def _pixel(a0, a1):
    mn = jnp.minimum(a0, a1)
    bw = (jnp.maximum(a0, a1) - mn) * (1.0 / NUM_BINS)
    pmin = mn - KR * bw
    inv = 1.0 / jnp.maximum(bw, EPS)
    p0 = (a0 - pmin) * inv
    p1 = (a1 - pmin) * inv
    g0 = p0.astype(jnp.int32).astype(jnp.float32)
    g1 = p1.astype(jnp.int32).astype(jnp.float32)
    f0 = p0 - g0
    f1 = p1 - g1
    zero = jnp.zeros_like(a0)
    w0 = (0.5 * f0 - 1.0) * (f0 * f0) + (2.0 / 3.0)
    w1 = (0.5 * f1 - 1.0) * (f1 * f1) + (2.0 / 3.0)
    n0 = jnp.where(g0 >= 1.0, w0 + _bsp(p1 - g0), zero)
    n1 = jnp.where(g1 >= 1.0, w1 + _bsp(p0 - g1), zero)
    v0 = jnp.maximum(2.0 - p0, 0.0)
    v1 = jnp.maximum(2.0 - p1, 0.0)
    q0 = jnp.maximum(1.0 - p0, 0.0)
    q1 = jnp.maximum(1.0 - p1, 0.0)
    cv = v0 * v0 * v0 + v1 * v1 * v1
    cq = q0 * q0 * q0 + q1 * q1 * q1
    hsum = jnp.maximum(2.0 - cv * (1.0 / 6.0) + cq * 0.5, EPS)
    rec = 1.0 / hsum
    return n0 * rec, n1 * rec


@functools.lru_cache(maxsize=1)
def _build():
    mesh = plsc.VectorSubcoreMesh(core_axis_name="c", subcore_axis_name="s")

    @functools.partial(
        pl.kernel,
        mesh=mesh,
        out_type=jax.ShapeDtypeStruct((TOT,), jnp.float32),
        scratch_types=[
            pltpu.VMEM((SPB,), jnp.float32),
            pltpu.VMEM((SPB,), jnp.float32),
            pltpu.VMEM((SPB,), jnp.float32),
            pltpu.VMEM((SPB,), jnp.float32),
            pltpu.SemaphoreType.DMA,
            pltpu.SemaphoreType.DMA,
            pltpu.SemaphoreType.DMA,
        ],
    )
    def _sc_kernel(img_hbm, out_hbm, v0, v1, o0, o1, sem_a, sem_b, sem_out):
        wid = lax.axis_index("c") * NS + lax.axis_index("s")
        b = wid // (NW // B)
        s = wid % (NW // B)
        off0 = b * (N * P) + s * SPB
        off1 = off0 + P
        HALF = SPB // 2
        ha0 = pltpu.async_copy(img_hbm.at[pl.ds(off0, HALF)],
                               v0.at[pl.ds(0, HALF)], sem_a)
        ha1 = pltpu.async_copy(img_hbm.at[pl.ds(off1, HALF)],
                               v1.at[pl.ds(0, HALF)], sem_a)
        hb0 = pltpu.async_copy(img_hbm.at[pl.ds(off0 + HALF, HALF)],
                               v0.at[pl.ds(HALF, HALF)], sem_b)
        hb1 = pltpu.async_copy(img_hbm.at[pl.ds(off1 + HALF, HALF)],
                               v1.at[pl.ds(HALF, HALF)], sem_b)

        def _body(i, carry):
            for u in range(UNROLL):
                sl = pl.ds((i * UNROLL + u) * L, L)
                r0, r1 = _pixel(v0[sl], v1[sl])
                o0[sl] = r0
                o1[sl] = r1
            return carry

        NIT = SPB // L // UNROLL
        ha0.wait()
        ha1.wait()
        lax.fori_loop(0, NIT // 2, _body, 0)
        ho0 = pltpu.async_copy(o0.at[pl.ds(0, HALF)],
                               out_hbm.at[pl.ds(off0, HALF)], sem_out)
        ho1 = pltpu.async_copy(o1.at[pl.ds(0, HALF)],
                               out_hbm.at[pl.ds(off1, HALF)], sem_out)
        hb0.wait()
        hb1.wait()
        lax.fori_loop(NIT // 2, NIT, _body, 0)
        ho2 = pltpu.async_copy(o0.at[pl.ds(HALF, HALF)],
                               out_hbm.at[pl.ds(off0 + HALF, HALF)], sem_out)
        ho3 = pltpu.async_copy(o1.at[pl.ds(HALF, HALF)],
                               out_hbm.at[pl.ds(off1 + HALF, HALF)], sem_out)
        ho0.wait()
        ho1.wait()
        ho2.wait()
        ho3.wait()

    return _sc_kernel


def kernel(images):
    flat = images.reshape(TOT)
    out = _build()(flat)
    return out.reshape(B, N, H, W)


# speedup vs baseline: 1.0198x; 1.0118x over previous
import functools
import jax
import jax.numpy as jnp
from jax import lax
from jax.experimental import pallas as pl
from jax.experimental.pallas import tpu as pltpu
from jax.experimental.pallas import tpu_sc as plsc

B = 4
N = 2
H = W = 384
P = H * W
TOT = B * N * P
NUM_BINS = 16
KR = 2
EPS = 1e-8

NC, NS, L = 2, 16, 16
NW = NC * NS
SPB = P // (NW // B)
NCHK = 6
CPX = SPB // NCHK
CVEC = CPX // L
UNROLL = 2


def _bsp(d):
    ad = jnp.abs(d)
    c1 = (0.5 * ad - 1.0) * (ad * ad) + (2.0 / 3.0)
    t = jnp.maximum(2.0 - ad, 0.0)
    c2 = t * t * t * (1.0 / 6.0)
    return jnp.where(ad < 1.0, c1, c2)


def _pixel(a0, a1):
    mn = jnp.minimum(a0, a1)
    bw = (jnp.maximum(a0, a1) - mn) * (1.0 / NUM_BINS)
    pmin = mn - KR * bw
    inv = 1.0 / jnp.maximum(bw, EPS)
    p0 = (a0 - pmin) * inv
    p1 = (a1 - pmin) * inv
    g0 = p0.astype(jnp.int32).astype(jnp.float32)
    g1 = p1.astype(jnp.int32).astype(jnp.float32)
    f0 = p0 - g0
    f1 = p1 - g1
    zero = jnp.zeros_like(a0)
    w0 = (0.5 * f0 - 1.0) * (f0 * f0) + (2.0 / 3.0)
    w1 = (0.5 * f1 - 1.0) * (f1 * f1) + (2.0 / 3.0)
    n0 = jnp.where(g0 >= 1.0, w0 + _bsp(p1 - g0), zero)
    n1 = jnp.where(g1 >= 1.0, w1 + _bsp(p0 - g1), zero)
    v0 = jnp.maximum(2.0 - p0, 0.0)
    v1 = jnp.maximum(2.0 - p1, 0.0)
    q0 = jnp.maximum(1.0 - p0, 0.0)
    q1 = jnp.maximum(1.0 - p1, 0.0)
    cv = v0 * v0 * v0 + v1 * v1 * v1
    cq = q0 * q0 * q0 + q1 * q1 * q1
    hsum = jnp.maximum(2.0 - cv * (1.0 / 6.0) + cq * 0.5, EPS)
    rec = 1.0 / hsum
    return n0 * rec, n1 * rec


@functools.lru_cache(maxsize=1)
def _build():
    mesh = plsc.VectorSubcoreMesh(core_axis_name="c", subcore_axis_name="s")

    @functools.partial(
        pl.kernel,
        mesh=mesh,
        out_type=jax.ShapeDtypeStruct((TOT,), jnp.float32),
        scratch_types=[
            pltpu.VMEM((SPB,), jnp.float32),
            pltpu.VMEM((SPB,), jnp.float32),
            pltpu.VMEM((SPB,), jnp.float32),
            pltpu.VMEM((SPB,), jnp.float32),
            pltpu.SemaphoreType.DMA,
            pltpu.SemaphoreType.DMA,
            pltpu.SemaphoreType.DMA,
        ],
    )
    def _sc_kernel(img_hbm, out_hbm, v0, v1, o0, o1, sem_a, sem_b, sem_out):
        wid = lax.axis_index("c") * NS + lax.axis_index("s")
        b = wid // (NW // B)
        s = wid % (NW // B)
        off0 = b * (N * P) + s * SPB
        off1 = off0 + P
        HALF = SPB // 2
        ha0 = pltpu.async_copy(img_hbm.at[pl.ds(off0, HALF)],
                               v0.at[pl.ds(0, HALF)], sem_a)
        ha1 = pltpu.async_copy(img_hbm.at[pl.ds(off1, HALF)],
                               v1.at[pl.ds(0, HALF)], sem_a)
        hb0 = pltpu.async_copy(img_hbm.at[pl.ds(off0 + HALF, HALF)],
                               v0.at[pl.ds(HALF, HALF)], sem_b)
        hb1 = pltpu.async_copy(img_hbm.at[pl.ds(off1 + HALF, HALF)],
                               v1.at[pl.ds(HALF, HALF)], sem_b)

        def _body(i, carry):
            for u in range(UNROLL):
                sl = pl.ds((i * UNROLL + u) * L, L)
                r0, r1 = _pixel(v0[sl], v1[sl])
                o0[sl] = r0
                o1[sl] = r1
            return carry

        NIT = SPB // L // UNROLL
        ha0.wait()
        ha1.wait()
        lax.fori_loop(0, NIT // 2, _body, 0)
        ho0 = pltpu.async_copy(o0.at[pl.ds(0, HALF)],
                               out_hbm.at[pl.ds(off0, HALF)], sem_out)
        ho1 = pltpu.async_copy(o1.at[pl.ds(0, HALF)],
                               out_hbm.at[pl.ds(off1, HALF)], sem_out)
        hb0.wait()
        hb1.wait()
        lax.fori_loop(NIT // 2, NIT, _body, 0)
        ho2 = pltpu.async_copy(o0.at[pl.ds(HALF, HALF)],
                               out_hbm.at[pl.ds(off0 + HALF, HALF)], sem_out)
        ho3 = pltpu.async_copy(o1.at[pl.ds(HALF, HALF)],
                               out_hbm.at[pl.ds(off1 + HALF, HALF)], sem_out)
        ho0.wait()
        ho1.wait()
        ho2.wait()
        ho3.wait()

    return _sc_kernel


def kernel(images):
    flat = images.reshape(TOT)
    out = _build()(flat)
    return out.reshape(B, N, H, W)


# speedup vs baseline: 1.2617x; 1.2372x over previous
import functools
import jax
import jax.numpy as jnp
from jax import lax
from jax.experimental import pallas as pl
from jax.experimental.pallas import tpu as pltpu
from jax.experimental.pallas import tpu_sc as plsc

B = 4
N = 2
H = W = 384
P = H * W
TOT = B * N * P
NUM_BINS = 16
KR = 2
EPS = 1e-8

NC, NS, L = 2, 16, 16
NW = NC * NS
PSC = P // 2
SPB = PSC // (NW // B)
UNROLL = 2
ROWS = P // 128
RSC = PSC // 128


def _bsp(d):
    ad = jnp.abs(d)
    c1 = (0.5 * ad - 1.0) * (ad * ad) + (2.0 / 3.0)
    t = jnp.maximum(2.0 - ad, 0.0)
    c2 = t * t * t * (1.0 / 6.0)
    return jnp.where(ad < 1.0, c1, c2)


def _pixel(a0, a1):
    mn = jnp.minimum(a0, a1)
    bw = (jnp.maximum(a0, a1) - mn) * (1.0 / NUM_BINS)
    pmin = mn - KR * bw
    inv = 1.0 / jnp.maximum(bw, EPS)
    p0 = (a0 - pmin) * inv
    p1 = (a1 - pmin) * inv
    g0 = p0.astype(jnp.int32).astype(jnp.float32)
    g1 = p1.astype(jnp.int32).astype(jnp.float32)
    f0 = p0 - g0
    f1 = p1 - g1
    zero = jnp.zeros_like(a0)
    w0 = (0.5 * f0 - 1.0) * (f0 * f0) + (2.0 / 3.0)
    w1 = (0.5 * f1 - 1.0) * (f1 * f1) + (2.0 / 3.0)
    n0 = jnp.where(g0 >= 1.0, w0 + _bsp(p1 - g0), zero)
    n1 = jnp.where(g1 >= 1.0, w1 + _bsp(p0 - g1), zero)
    v0 = jnp.maximum(2.0 - p0, 0.0)
    v1 = jnp.maximum(2.0 - p1, 0.0)
    q0 = jnp.maximum(1.0 - p0, 0.0)
    q1 = jnp.maximum(1.0 - p1, 0.0)
    cv = v0 * v0 * v0 + v1 * v1 * v1
    cq = q0 * q0 * q0 + q1 * q1 * q1
    hsum = jnp.maximum(2.0 - cv * (1.0 / 6.0) + cq * 0.5, EPS)
    rec = 1.0 / hsum
    return n0 * rec, n1 * rec


@functools.lru_cache(maxsize=1)
def _build():
    mesh = plsc.VectorSubcoreMesh(core_axis_name="c", subcore_axis_name="s")

    @functools.partial(
        pl.kernel,
        mesh=mesh,
        out_type=jax.ShapeDtypeStruct((B * N * PSC,), jnp.float32),
        scratch_types=[
            pltpu.VMEM((SPB,), jnp.float32),
            pltpu.VMEM((SPB,), jnp.float32),
            pltpu.VMEM((SPB,), jnp.float32),
            pltpu.VMEM((SPB,), jnp.float32),
            pltpu.SemaphoreType.DMA,
            pltpu.SemaphoreType.DMA,
            pltpu.SemaphoreType.DMA,
        ],
    )
    def _sc_kernel(img_hbm, out_hbm, v0, v1, o0, o1, sem_a, sem_b, sem_out):
        wid = lax.axis_index("c") * NS + lax.axis_index("s")
        b = wid // (NW // B)
        s = wid % (NW // B)
        off0 = b * (N * P) + s * SPB
        off1 = off0 + P
        oo0 = b * (N * PSC) + s * SPB
        oo1 = oo0 + PSC
        HALF = SPB // 2
        ha0 = pltpu.async_copy(img_hbm.at[pl.ds(off0, HALF)],
                               v0.at[pl.ds(0, HALF)], sem_a)
        ha1 = pltpu.async_copy(img_hbm.at[pl.ds(off1, HALF)],
                               v1.at[pl.ds(0, HALF)], sem_a)
        hb0 = pltpu.async_copy(img_hbm.at[pl.ds(off0 + HALF, HALF)],
                               v0.at[pl.ds(HALF, HALF)], sem_b)
        hb1 = pltpu.async_copy(img_hbm.at[pl.ds(off1 + HALF, HALF)],
                               v1.at[pl.ds(HALF, HALF)], sem_b)

        def _body(i, carry):
            for u in range(UNROLL):
                sl = pl.ds((i * UNROLL + u) * L, L)
                r0, r1 = _pixel(v0[sl], v1[sl])
                o0[sl] = r0
                o1[sl] = r1
            return carry

        NIT = SPB // L // UNROLL
        ha0.wait()
        ha1.wait()
        lax.fori_loop(0, NIT // 2, _body, 0)
        ho0 = pltpu.async_copy(o0.at[pl.ds(0, HALF)],
                               out_hbm.at[pl.ds(oo0, HALF)], sem_out)
        ho1 = pltpu.async_copy(o1.at[pl.ds(0, HALF)],
                               out_hbm.at[pl.ds(oo1, HALF)], sem_out)
        hb0.wait()
        hb1.wait()
        lax.fori_loop(NIT // 2, NIT, _body, 0)
        ho2 = pltpu.async_copy(o0.at[pl.ds(HALF, HALF)],
                               out_hbm.at[pl.ds(oo0 + HALF, HALF)], sem_out)
        ho3 = pltpu.async_copy(o1.at[pl.ds(HALF, HALF)],
                               out_hbm.at[pl.ds(oo1 + HALF, HALF)], sem_out)
        ho0.wait()
        ho1.wait()
        ho2.wait()
        ho3.wait()

    return _sc_kernel


def _tc_body(x_ref, o_ref):
    r0, r1 = _pixel(x_ref[0, 0], x_ref[0, 1])
    o_ref[0, 0] = r0
    o_ref[0, 1] = r1


def _tc_half(x4):
    return pl.pallas_call(
        _tc_body,
        grid=(B,),
        in_specs=[pl.BlockSpec((1, N, ROWS - RSC, 128),
                               lambda i: (i, 0, 1, 0))],
        out_specs=pl.BlockSpec((1, N, ROWS - RSC, 128),
                               lambda i: (i, 0, 0, 0)),
        out_shape=jax.ShapeDtypeStruct((B, N, ROWS - RSC, 128), jnp.float32),
    )(x4)


def kernel(images):
    flat = images.reshape(TOT)
    sc_out = _build()(flat)
    tc_out = _tc_half(images.reshape(B, N, ROWS, 128))
    sc4 = sc_out.reshape(B, N, RSC, 128)
    return jnp.concatenate([sc4, tc_out], axis=2).reshape(B, N, H, W)


# speedup vs baseline: 1.3518x; 1.0714x over previous
import functools
import jax
import jax.numpy as jnp
from jax import lax
from jax.experimental import pallas as pl
from jax.experimental.pallas import tpu as pltpu
from jax.experimental.pallas import tpu_sc as plsc

B = 4
N = 2
H = W = 384
P = H * W
TOT = B * N * P
NUM_BINS = 16
KR = 2
EPS = 1e-8

NC, NS, L = 2, 16, 16
NW = NC * NS
PSC = 3 * P // 8
SPB = PSC // (NW // B)
UNROLL = 2
ROWS = P // 128
RSC = PSC // 128


def _bsp(d):
    ad = jnp.abs(d)
    c1 = (0.5 * ad - 1.0) * (ad * ad) + (2.0 / 3.0)
    t = jnp.maximum(2.0 - ad, 0.0)
    c2 = t * t * t * (1.0 / 6.0)
    return jnp.where(ad < 1.0, c1, c2)


def _pixel(a0, a1):
    mn = jnp.minimum(a0, a1)
    bw = (jnp.maximum(a0, a1) - mn) * (1.0 / NUM_BINS)
    pmin = mn - KR * bw
    inv = 1.0 / jnp.maximum(bw, EPS)
    p0 = (a0 - pmin) * inv
    p1 = (a1 - pmin) * inv
    g0 = p0.astype(jnp.int32).astype(jnp.float32)
    g1 = p1.astype(jnp.int32).astype(jnp.float32)
    f0 = p0 - g0
    f1 = p1 - g1
    zero = jnp.zeros_like(a0)
    w0 = (0.5 * f0 - 1.0) * (f0 * f0) + (2.0 / 3.0)
    w1 = (0.5 * f1 - 1.0) * (f1 * f1) + (2.0 / 3.0)
    n0 = jnp.where(g0 >= 1.0, w0 + _bsp(p1 - g0), zero)
    n1 = jnp.where(g1 >= 1.0, w1 + _bsp(p0 - g1), zero)
    v0 = jnp.maximum(2.0 - p0, 0.0)
    v1 = jnp.maximum(2.0 - p1, 0.0)
    q0 = jnp.maximum(1.0 - p0, 0.0)
    q1 = jnp.maximum(1.0 - p1, 0.0)
    cv = v0 * v0 * v0 + v1 * v1 * v1
    cq = q0 * q0 * q0 + q1 * q1 * q1
    hsum = jnp.maximum(2.0 - cv * (1.0 / 6.0) + cq * 0.5, EPS)
    rec = 1.0 / hsum
    return n0 * rec, n1 * rec


@functools.lru_cache(maxsize=1)
def _build():
    mesh = plsc.VectorSubcoreMesh(core_axis_name="c", subcore_axis_name="s")

    @functools.partial(
        pl.kernel,
        mesh=mesh,
        out_type=jax.ShapeDtypeStruct((B * N * PSC,), jnp.float32),
        scratch_types=[
            pltpu.VMEM((SPB,), jnp.float32),
            pltpu.VMEM((SPB,), jnp.float32),
            pltpu.VMEM((SPB,), jnp.float32),
            pltpu.VMEM((SPB,), jnp.float32),
            pltpu.SemaphoreType.DMA,
            pltpu.SemaphoreType.DMA,
            pltpu.SemaphoreType.DMA,
        ],
    )
    def _sc_kernel(img_hbm, out_hbm, v0, v1, o0, o1, sem_a, sem_b, sem_out):
        wid = lax.axis_index("c") * NS + lax.axis_index("s")
        b = wid // (NW // B)
        s = wid % (NW // B)
        off0 = b * (N * P) + s * SPB
        off1 = off0 + P
        oo0 = b * (N * PSC) + s * SPB
        oo1 = oo0 + PSC
        HALF = SPB // 2
        ha0 = pltpu.async_copy(img_hbm.at[pl.ds(off0, HALF)],
                               v0.at[pl.ds(0, HALF)], sem_a)
        ha1 = pltpu.async_copy(img_hbm.at[pl.ds(off1, HALF)],
                               v1.at[pl.ds(0, HALF)], sem_a)
        hb0 = pltpu.async_copy(img_hbm.at[pl.ds(off0 + HALF, HALF)],
                               v0.at[pl.ds(HALF, HALF)], sem_b)
        hb1 = pltpu.async_copy(img_hbm.at[pl.ds(off1 + HALF, HALF)],
                               v1.at[pl.ds(HALF, HALF)], sem_b)

        def _body(i, carry):
            for u in range(UNROLL):
                sl = pl.ds((i * UNROLL + u) * L, L)
                r0, r1 = _pixel(v0[sl], v1[sl])
                o0[sl] = r0
                o1[sl] = r1
            return carry

        NIT = SPB // L // UNROLL
        ha0.wait()
        ha1.wait()
        lax.fori_loop(0, NIT // 2, _body, 0)
        ho0 = pltpu.async_copy(o0.at[pl.ds(0, HALF)],
                               out_hbm.at[pl.ds(oo0, HALF)], sem_out)
        ho1 = pltpu.async_copy(o1.at[pl.ds(0, HALF)],
                               out_hbm.at[pl.ds(oo1, HALF)], sem_out)
        hb0.wait()
        hb1.wait()
        lax.fori_loop(NIT // 2, NIT, _body, 0)
        ho2 = pltpu.async_copy(o0.at[pl.ds(HALF, HALF)],
                               out_hbm.at[pl.ds(oo0 + HALF, HALF)], sem_out)
        ho3 = pltpu.async_copy(o1.at[pl.ds(HALF, HALF)],
                               out_hbm.at[pl.ds(oo1 + HALF, HALF)], sem_out)
        ho0.wait()
        ho1.wait()
        ho2.wait()
        ho3.wait()

    return _sc_kernel


def _tc_body(x_ref, o_ref):
    r0, r1 = _pixel(x_ref[0, 0], x_ref[0, 1])
    o_ref[0, 0] = r0
    o_ref[0, 1] = r1


def _tc_half(x4):
    return pl.pallas_call(
        _tc_body,
        grid=(B,),
        in_specs=[pl.BlockSpec((1, N, ROWS - RSC, 128),
                               lambda i: (i, 0, 1, 0))],
        out_specs=pl.BlockSpec((1, N, ROWS - RSC, 128),
                               lambda i: (i, 0, 0, 0)),
        out_shape=jax.ShapeDtypeStruct((B, N, ROWS - RSC, 128), jnp.float32),
    )(x4)


def kernel(images):
    flat = images.reshape(TOT)
    sc_out = _build()(flat)
    tc_out = _tc_half(images.reshape(B, N, ROWS, 128))
    sc4 = sc_out.reshape(B, N, RSC, 128)
    return jnp.concatenate([sc4, tc_out], axis=2).reshape(B, N, H, W)
